# baseline (speedup 1.0000x reference)
import functools

import jax
import jax.numpy as jnp
from jax.experimental import pallas as pl
from jax.experimental.pallas import tpu as pltpu

D = 2048
BJ = 256
NJ = D // BJ


def _gate_kernel(x_ref, wgr_ref, bgr_ref, wgi_ref, bgi_ref,
                 idx_ref, probs_ref, mir_ref, mii_ref):
    x = x_ref[...]
    dn = (((1,), (1,)), ((), ()))
    sr = jax.nn.relu(
        jax.lax.dot_general(x, wgr_ref[...], dn,
                            preferred_element_type=jnp.float32) + bgr_ref[...])
    si = jax.nn.relu(
        jax.lax.dot_general(x, wgi_ref[...], dn,
                            preferred_element_type=jnp.float32) + bgi_ref[...])

    def route(s):
        a, b, c, d = s[:, 0:1], s[:, 1:2], s[:, 2:3], s[:, 3:4]
        i1 = jnp.where(a >= b, 0, 1).astype(jnp.int32)
        s1 = jnp.maximum(a, b)
        i2 = jnp.where(c >= d, 2, 3).astype(jnp.int32)
        s2 = jnp.maximum(c, d)
        m = jnp.maximum(s1, s2)
        e1 = jnp.exp(s1 - m)
        e2 = jnp.exp(s2 - m)
        denom = e1 + e2
        p1 = e1 / denom
        p2 = e2 / denom
        mi = jnp.where(p1 >= p2, 0, 1).astype(jnp.int32)
        return i1, i2, p1, p2, mi

    ir1, ir2, pr1, pr2, mir = route(sr)
    ii1, ii2, pi1, pi2, mii = route(si)

    idx_ref[...] = jnp.concatenate([ir1, ir2, ii1, ii2], axis=1)
    probs_ref[...] = jnp.concatenate([pr1, pr2, pi1, pi2], axis=1)
    mir_ref[...] = mir
    mii_ref[...] = mii


def _combine_kernel(idx_ref, probs_ref, x_ref, wr_ref, wi_ref,
                    br_ref, bi_ref, out_ref):
    k = pl.program_id(1)
    pr = probs_ref[k]
    pi = probs_ref[2 + k]
    x = x_ref[...]
    dn = (((1,), (1,)), ((), ()))
    yr = jax.lax.dot_general(x, wr_ref[0], dn,
                             preferred_element_type=jnp.float32)
    yi = jax.lax.dot_general(x, wi_ref[0], dn,
                             preferred_element_type=jnp.float32)
    contrib = pr * (yr + br_ref[0]) + pi * (yi + bi_ref[0])

    @pl.when(k == 0)
    def _init():
        out_ref[...] = contrib

    @pl.when(k == 1)
    def _acc():
        out_ref[...] += contrib


@jax.jit
def kernel(rgb_local, ir_local, W_gate_rgb, b_gate_rgb, W_gate_ir, b_gate_ir,
           W_exp_rgb, b_exp_rgb, W_exp_ir, b_exp_ir):
    B = rgb_local.shape[0]
    x = jnp.concatenate(
        [rgb_local.reshape(B, -1), ir_local.reshape(B, -1)], axis=1)

    idx, probs, max_idx_rgb, max_idx_ir = pl.pallas_call(
        _gate_kernel,
        out_shape=(
            jax.ShapeDtypeStruct((1, 4), jnp.int32),
            jax.ShapeDtypeStruct((1, 4), jnp.float32),
            jax.ShapeDtypeStruct((1, 1), jnp.int32),
            jax.ShapeDtypeStruct((1, 1), jnp.int32),
        ),
    )(x, W_gate_rgb, b_gate_rgb.reshape(1, 4), W_gate_ir,
      b_gate_ir.reshape(1, 4))

    grid_spec = pltpu.PrefetchScalarGridSpec(
        num_scalar_prefetch=2,
        grid=(NJ, 2),
        in_specs=[
            pl.BlockSpec((1, D), lambda j, k, idx, p: (0, 0)),
            pl.BlockSpec((1, BJ, D), lambda j, k, idx, p: (idx[k], j, 0)),
            pl.BlockSpec((1, BJ, D), lambda j, k, idx, p: (idx[2 + k], j, 0)),
            pl.BlockSpec((1, 1, BJ), lambda j, k, idx, p: (idx[k], 0, j)),
            pl.BlockSpec((1, 1, BJ), lambda j, k, idx, p: (idx[2 + k], 0, j)),
        ],
        out_specs=pl.BlockSpec((1, BJ), lambda j, k, idx, p: (0, j)),
    )
    combined = pl.pallas_call(
        _combine_kernel,
        grid_spec=grid_spec,
        out_shape=jax.ShapeDtypeStruct((1, D), jnp.float32),
        compiler_params=pltpu.CompilerParams(
            dimension_semantics=("arbitrary", "arbitrary")),
    )(idx.reshape(4), probs.reshape(4), x, W_exp_rgb, W_exp_ir,
      b_exp_rgb.reshape(4, 1, D), b_exp_ir.reshape(4, 1, D))

    return (combined, max_idx_rgb.reshape(1), max_idx_ir.reshape(1))


# speedup vs baseline: 1.3197x; 1.3197x over previous
import jax
import jax.numpy as jnp
from jax.experimental import pallas as pl
from jax.experimental.pallas import tpu as pltpu

D = 2048
BJ = 512
NJ = D // BJ
NBUF = 3


def _fused_kernel(x_ref, wgr_ref, bgr_ref, wgi_ref, bgi_ref,
                  wr_hbm, wi_hbm, br_ref, bi_ref,
                  out_ref, mir_ref, mii_ref,
                  wr_buf, wi_buf, rsem, isem):
    x = x_ref[...]

    def route(wg_ref, bg_ref):
        s = [jnp.maximum(jnp.sum(x * wg_ref[e:e + 1, :]) + bg_ref[e], 0.0)
             for e in range(4)]
        i1 = jnp.where(s[0] >= s[1], 0, 1)
        s1 = jnp.maximum(s[0], s[1])
        i2 = jnp.where(s[2] >= s[3], 2, 3)
        s2 = jnp.maximum(s[2], s[3])
        m = jnp.maximum(s1, s2)
        e1 = jnp.exp(jnp.broadcast_to(s1 - m, (1, 1)))
        e2 = jnp.exp(jnp.broadcast_to(s2 - m, (1, 1)))
        denom = e1 + e2
        p1 = e1 / denom
        p2 = e2 / denom
        mi = jnp.where(p1 >= p2, 0, 1).astype(jnp.int32)
        return i1, i2, p1, p2, mi

    ir1, ir2, pr1, pr2, mir = route(wgr_ref, bgr_ref)
    ii1, ii2, pi1, pi2, mii = route(wgi_ref, bgi_ref)
    mir_ref[...] = mir
    mii_ref[...] = mii

    lanes = jax.lax.broadcasted_iota(jnp.int32, (4, 1), 0)
    w_r = (jnp.where(lanes == ir1, pr1, 0.0) +
           jnp.where(lanes == ir2, pr2, 0.0))
    w_i = (jnp.where(lanes == ii1, pi1, 0.0) +
           jnp.where(lanes == ii2, pi2, 0.0))
    out_ref[...] = (jnp.sum(w_r * br_ref[...], axis=0, keepdims=True) +
                    jnp.sum(w_i * bi_ref[...], axis=0, keepdims=True))

    steps = [(j, k) for j in range(NJ) for k in range(2)]
    e_r = [ir1, ir2]
    e_i = [ii1, ii2]
    p_r = [pr1, pr2]
    p_i = [pi1, pi2]

    def copies(t, b):
        j, k = steps[t]
        src_r = wr_hbm.at[e_r[k], pl.ds(j * BJ, BJ), :]
        src_i = wi_hbm.at[e_i[k], pl.ds(j * BJ, BJ), :]
        return (pltpu.make_async_copy(src_r, wr_buf.at[b], rsem.at[b]),
                pltpu.make_async_copy(src_i, wi_buf.at[b], isem.at[b]))

    for t in range(min(NBUF, len(steps))):
        cr, ci = copies(t, t % NBUF)
        cr.start()
        ci.start()

    dn = (((1,), (1,)), ((), ()))
    for t, (j, k) in enumerate(steps):
        b = t % NBUF
        cr, ci = copies(t, b)
        cr.wait()
        ci.wait()
        yr = jax.lax.dot_general(x, wr_buf[b], dn,
                                 preferred_element_type=jnp.float32)
        yi = jax.lax.dot_general(x, wi_buf[b], dn,
                                 preferred_element_type=jnp.float32)
        out_ref[:, pl.ds(j * BJ, BJ)] += p_r[k] * yr + p_i[k] * yi
        nxt = t + NBUF
        if nxt < len(steps):
            nr, ni = copies(nxt, nxt % NBUF)
            nr.start()
            ni.start()


@jax.jit
def kernel(rgb_local, ir_local, W_gate_rgb, b_gate_rgb, W_gate_ir, b_gate_ir,
           W_exp_rgb, b_exp_rgb, W_exp_ir, b_exp_ir):
    B = rgb_local.shape[0]
    x = jnp.concatenate(
        [rgb_local.reshape(B, -1), ir_local.reshape(B, -1)], axis=1)

    combined, max_idx_rgb, max_idx_ir = pl.pallas_call(
        _fused_kernel,
        in_specs=[
            pl.BlockSpec(memory_space=pltpu.VMEM),
            pl.BlockSpec(memory_space=pltpu.VMEM),
            pl.BlockSpec(memory_space=pltpu.SMEM),
            pl.BlockSpec(memory_space=pltpu.VMEM),
            pl.BlockSpec(memory_space=pltpu.SMEM),
            pl.BlockSpec(memory_space=pl.ANY),
            pl.BlockSpec(memory_space=pl.ANY),
            pl.BlockSpec(memory_space=pltpu.VMEM),
            pl.BlockSpec(memory_space=pltpu.VMEM),
        ],
        out_specs=(
            pl.BlockSpec(memory_space=pltpu.VMEM),
            pl.BlockSpec(memory_space=pltpu.VMEM),
            pl.BlockSpec(memory_space=pltpu.VMEM),
        ),
        out_shape=(
            jax.ShapeDtypeStruct((1, D), jnp.float32),
            jax.ShapeDtypeStruct((1, 1), jnp.int32),
            jax.ShapeDtypeStruct((1, 1), jnp.int32),
        ),
        scratch_shapes=[
            pltpu.VMEM((NBUF, BJ, D), jnp.float32),
            pltpu.VMEM((NBUF, BJ, D), jnp.float32),
            pltpu.SemaphoreType.DMA((NBUF,)),
            pltpu.SemaphoreType.DMA((NBUF,)),
        ],
    )(x, W_gate_rgb, b_gate_rgb, W_gate_ir, b_gate_ir,
      W_exp_rgb, W_exp_ir, b_exp_rgb, b_exp_ir)

    return (combined, max_idx_rgb.reshape(1), max_idx_ir.reshape(1))


# speedup vs baseline: 1.3281x; 1.0064x over previous
import jax
import jax.numpy as jnp
from jax.experimental import pallas as pl
from jax.experimental.pallas import tpu as pltpu

D = 2048
BJ = 256
NJ = D // BJ
NBUF = 6


def _fused_kernel(x_ref, wgr_ref, bgr_ref, wgi_ref, bgi_ref,
                  wr_hbm, wi_hbm, br_ref, bi_ref,
                  out_ref, mir_ref, mii_ref,
                  wr_buf, wi_buf, rsem, isem):
    x = x_ref[...]

    def route(wg_ref, bg_ref):
        s = [jnp.maximum(jnp.sum(x * wg_ref[e:e + 1, :]) + bg_ref[e], 0.0)
             for e in range(4)]
        i1 = jnp.where(s[0] >= s[1], 0, 1)
        s1 = jnp.maximum(s[0], s[1])
        i2 = jnp.where(s[2] >= s[3], 2, 3)
        s2 = jnp.maximum(s[2], s[3])
        m = jnp.maximum(s1, s2)
        e1 = jnp.exp(jnp.broadcast_to(s1 - m, (1, 1)))
        e2 = jnp.exp(jnp.broadcast_to(s2 - m, (1, 1)))
        denom = e1 + e2
        p1 = e1 / denom
        p2 = e2 / denom
        mi = jnp.where(p1 >= p2, 0, 1).astype(jnp.int32)
        return i1, i2, p1, p2, mi

    ir1, ir2, pr1, pr2, mir = route(wgr_ref, bgr_ref)
    ii1, ii2, pi1, pi2, mii = route(wgi_ref, bgi_ref)
    mir_ref[...] = mir
    mii_ref[...] = mii

    lanes = jax.lax.broadcasted_iota(jnp.int32, (4, 1), 0)
    w_r = (jnp.where(lanes == ir1, pr1, 0.0) +
           jnp.where(lanes == ir2, pr2, 0.0))
    w_i = (jnp.where(lanes == ii1, pi1, 0.0) +
           jnp.where(lanes == ii2, pi2, 0.0))
    out_ref[...] = (jnp.sum(w_r * br_ref[...], axis=0, keepdims=True) +
                    jnp.sum(w_i * bi_ref[...], axis=0, keepdims=True))

    steps = [(j, k) for j in range(NJ) for k in range(2)]
    e_r = [ir1, ir2]
    e_i = [ii1, ii2]
    p_r = [pr1, pr2]
    p_i = [pi1, pi2]

    def copies(t, b):
        j, k = steps[t]
        src_r = wr_hbm.at[e_r[k], pl.ds(j * BJ, BJ), :]
        src_i = wi_hbm.at[e_i[k], pl.ds(j * BJ, BJ), :]
        return (pltpu.make_async_copy(src_r, wr_buf.at[b], rsem.at[b]),
                pltpu.make_async_copy(src_i, wi_buf.at[b], isem.at[b]))

    for t in range(min(NBUF, len(steps))):
        cr, ci = copies(t, t % NBUF)
        cr.start()
        ci.start()

    dn = (((1,), (1,)), ((), ()))
    for t, (j, k) in enumerate(steps):
        b = t % NBUF
        cr, ci = copies(t, b)
        cr.wait()
        ci.wait()
        yr = jax.lax.dot_general(x, wr_buf[b], dn,
                                 preferred_element_type=jnp.float32)
        yi = jax.lax.dot_general(x, wi_buf[b], dn,
                                 preferred_element_type=jnp.float32)
        out_ref[:, pl.ds(j * BJ, BJ)] += p_r[k] * yr + p_i[k] * yi
        nxt = t + NBUF
        if nxt < len(steps):
            nr, ni = copies(nxt, nxt % NBUF)
            nr.start()
            ni.start()


@jax.jit
def kernel(rgb_local, ir_local, W_gate_rgb, b_gate_rgb, W_gate_ir, b_gate_ir,
           W_exp_rgb, b_exp_rgb, W_exp_ir, b_exp_ir):
    B = rgb_local.shape[0]
    x = jnp.concatenate(
        [rgb_local.reshape(B, -1), ir_local.reshape(B, -1)], axis=1)

    combined, max_idx_rgb, max_idx_ir = pl.pallas_call(
        _fused_kernel,
        in_specs=[
            pl.BlockSpec(memory_space=pltpu.VMEM),
            pl.BlockSpec(memory_space=pltpu.VMEM),
            pl.BlockSpec(memory_space=pltpu.SMEM),
            pl.BlockSpec(memory_space=pltpu.VMEM),
            pl.BlockSpec(memory_space=pltpu.SMEM),
            pl.BlockSpec(memory_space=pl.ANY),
            pl.BlockSpec(memory_space=pl.ANY),
            pl.BlockSpec(memory_space=pltpu.VMEM),
            pl.BlockSpec(memory_space=pltpu.VMEM),
        ],
        out_specs=(
            pl.BlockSpec(memory_space=pltpu.VMEM),
            pl.BlockSpec(memory_space=pltpu.VMEM),
            pl.BlockSpec(memory_space=pltpu.VMEM),
        ),
        out_shape=(
            jax.ShapeDtypeStruct((1, D), jnp.float32),
            jax.ShapeDtypeStruct((1, 1), jnp.int32),
            jax.ShapeDtypeStruct((1, 1), jnp.int32),
        ),
        scratch_shapes=[
            pltpu.VMEM((NBUF, BJ, D), jnp.float32),
            pltpu.VMEM((NBUF, BJ, D), jnp.float32),
            pltpu.SemaphoreType.DMA((NBUF,)),
            pltpu.SemaphoreType.DMA((NBUF,)),
        ],
    )(x, W_gate_rgb, b_gate_rgb, W_gate_ir, b_gate_ir,
      W_exp_rgb, W_exp_ir, b_exp_rgb, b_exp_ir)

    return (combined, max_idx_rgb.reshape(1), max_idx_ir.reshape(1))
